# Initial kernel scaffold; baseline (speedup 1.0000x reference)
import jax
import jax.numpy as jnp
from jax.experimental import pallas as pl


def kernel(user_id, item_id, neg_items, user_table, item_table, W1, b1, W2, b2):
    raise NotImplementedError("write your pallas kernel here")


# speedup vs baseline: 3.5778x; 3.5778x over previous
import functools

import jax
import jax.numpy as jnp
from jax import lax
from jax.experimental import pallas as pl
from jax.experimental.pallas import tpu as pltpu
from jax.experimental.pallas import tpu_sc as plsc


def _sc_gather(user_table, item_table, uid, cid, B, NCAND, D):
    NW = 32
    u_per_w = B // NW
    c_per_w = (B * NCAND) // NW
    CHUNK = 512
    n_chunks = c_per_w // CHUNK

    mesh = plsc.VectorSubcoreMesh(core_axis_name="c", subcore_axis_name="s")

    @functools.partial(
        pl.kernel,
        out_type=(
            jax.ShapeDtypeStruct((B, D), jnp.float32),
            jax.ShapeDtypeStruct((B * NCAND, D), jnp.float32),
        ),
        mesh=mesh,
        scratch_types=[
            pltpu.VMEM((CHUNK,), jnp.int32),
            pltpu.VMEM((CHUNK, D), jnp.float32),
            pltpu.VMEM((CHUNK,), jnp.int32),
            pltpu.VMEM((CHUNK, D), jnp.float32),
            pltpu.SemaphoreType.DMA,
            pltpu.SemaphoreType.DMA,
        ],
        compiler_params=pltpu.CompilerParams(use_tc_tiling_on_sc=False),
    )
    def k(ut, it, uid_h, cid_h, uout, cout, idx0, buf0, idx1, buf1, sem0, sem1):
        wid = lax.axis_index("s") * 2 + lax.axis_index("c")
        ubase = wid * u_per_w
        cbase = wid * c_per_w
        idxs = (idx0, idx1)
        bufs = (buf0, buf1)
        sems = (sem0, sem1)

        pltpu.sync_copy(uid_h.at[pl.ds(ubase, CHUNK)], idx0)
        ug = pltpu.async_copy(ut.at[idx0], buf0, sem0)
        pltpu.sync_copy(cid_h.at[pl.ds(cbase, CHUNK)], idx1)
        cg = pltpu.async_copy(it.at[idx1], buf1, sem1)
        ug.wait()
        pltpu.sync_copy(buf0, uout.at[pl.ds(ubase, CHUNK)])
        pending = cg
        for j in range(n_chunks):
            nxt = (j + 1) % 2
            if j + 1 < n_chunks:
                pltpu.sync_copy(
                    cid_h.at[pl.ds(cbase + (j + 1) * CHUNK, CHUNK)], idxs[nxt]
                )
                nxt_copy = pltpu.async_copy(it.at[idxs[nxt]], bufs[nxt], sems[nxt])
            pending.wait()
            pltpu.sync_copy(bufs[j % 2], cout.at[pl.ds(cbase + j * CHUNK, CHUNK)])
            if j + 1 < n_chunks:
                pending = nxt_copy

    return k(user_table, item_table, uid, cid)


def _mlp_body(u_ref, c_ref, w1_ref, b1_ref, w2t_ref, b2_ref, out_ref, *, bu, ncand, d):
    u = u_ref[...]
    c = c_ref[...]
    w1u = w1_ref[:d, :]
    w1i = w1_ref[d:, :]
    a = jnp.dot(u, w1u, preferred_element_type=jnp.float32)
    t = jnp.dot(c, w1i, preferred_element_type=jnp.float32)
    t = t.reshape(bu, ncand, d)
    h = jnp.tanh(t + a[:, None, :] + b1_ref[...][None, :, :])
    logits = jnp.sum(h * w2t_ref[...][None, :, :], axis=-1) + b2_ref[0, 0]
    out_ref[...] = jax.nn.sigmoid(logits)


def _tc_mlp(u_emb, c_emb, W1, b1, W2, b2, B, NCAND, D, interpret=False):
    BU = 512
    grid = (B // BU,)
    body = functools.partial(_mlp_body, bu=BU, ncand=NCAND, d=D)
    return pl.pallas_call(
        body,
        grid=grid,
        in_specs=[
            pl.BlockSpec((BU, D), lambda i: (i, 0)),
            pl.BlockSpec((BU * NCAND, D), lambda i: (i, 0)),
            pl.BlockSpec((2 * D, D), lambda i: (0, 0)),
            pl.BlockSpec((1, D), lambda i: (0, 0)),
            pl.BlockSpec((1, D), lambda i: (0, 0)),
            pl.BlockSpec((1, 1), lambda i: (0, 0)),
        ],
        out_specs=pl.BlockSpec((BU, NCAND), lambda i: (i, 0)),
        out_shape=jax.ShapeDtypeStruct((B, NCAND), jnp.float32),
        interpret=interpret,
    )(u_emb, c_emb, W1, b1.reshape(1, D), W2.reshape(1, D), b2.reshape(1, 1))


def kernel(user_id, item_id, neg_items, user_table, item_table, W1, b1, W2, b2):
    B = user_id.shape[0]
    NCAND = neg_items.shape[1] + 1
    D = user_table.shape[1]
    uid = user_id.astype(jnp.int32)
    cid = jnp.concatenate(
        [item_id.reshape(-1, 1), neg_items], axis=-1
    ).reshape(-1).astype(jnp.int32)
    u_emb, c_emb = _sc_gather(user_table, item_table, uid, cid, B, NCAND, D)
    return _tc_mlp(u_emb, c_emb, W1, b1, W2, b2, B, NCAND, D)


# speedup vs baseline: 3.6266x; 1.0137x over previous
import functools

import jax
import jax.numpy as jnp
from jax import lax
from jax.experimental import pallas as pl
from jax.experimental.pallas import tpu as pltpu
from jax.experimental.pallas import tpu_sc as plsc


def _sc_gather(user_table, item_table, uid, cid, B, NCAND, D):
    NW = 32
    u_per_w = B // NW
    c_per_w = (B * NCAND) // NW
    CHUNK = 512
    n_chunks = c_per_w // CHUNK

    mesh = plsc.VectorSubcoreMesh(core_axis_name="c", subcore_axis_name="s")

    @functools.partial(
        pl.kernel,
        out_type=(
            jax.ShapeDtypeStruct((B, D), jnp.float32),
            jax.ShapeDtypeStruct((B * NCAND, D), jnp.float32),
        ),
        mesh=mesh,
        scratch_types=[
            pltpu.VMEM((u_per_w,), jnp.int32),
            pltpu.VMEM((u_per_w, D), jnp.float32),
            pltpu.VMEM((CHUNK,), jnp.int32),
            pltpu.VMEM((CHUNK, D), jnp.float32),
            pltpu.VMEM((CHUNK,), jnp.int32),
            pltpu.VMEM((CHUNK, D), jnp.float32),
            pltpu.SemaphoreType.DMA,
            pltpu.SemaphoreType.DMA,
            pltpu.SemaphoreType.DMA,
        ],
        compiler_params=pltpu.CompilerParams(use_tc_tiling_on_sc=False),
    )
    def k(ut, it, uid_h, cid_h, uout, cout,
          uidx, ubuf, idx0, buf0, idx1, buf1, usem, sem0, sem1):
        wid = lax.axis_index("s") * 2 + lax.axis_index("c")
        ubase = wid * u_per_w
        cbase = wid * c_per_w
        idxs = (idx0, idx1)
        bufs = (buf0, buf1)
        sems = (sem0, sem1)

        pltpu.sync_copy(uid_h.at[pl.ds(ubase, u_per_w)], uidx)
        ug = pltpu.async_copy(ut.at[uidx], ubuf, usem)
        pltpu.sync_copy(cid_h.at[pl.ds(cbase, CHUNK)], idx0)
        cg = pltpu.async_copy(it.at[idx0], buf0, sem0)
        ug.wait()
        pltpu.sync_copy(ubuf, uout.at[pl.ds(ubase, u_per_w)])
        pending = cg
        for j in range(n_chunks):
            nxt = (j + 1) % 2
            if j + 1 < n_chunks:
                pltpu.sync_copy(
                    cid_h.at[pl.ds(cbase + (j + 1) * CHUNK, CHUNK)], idxs[nxt]
                )
                nxt_copy = pltpu.async_copy(it.at[idxs[nxt]], bufs[nxt], sems[nxt])
            pending.wait()
            pltpu.sync_copy(bufs[j % 2], cout.at[pl.ds(cbase + j * CHUNK, CHUNK)])
            if j + 1 < n_chunks:
                pending = nxt_copy

    return k(user_table, item_table, uid, cid)


def _mlp_body(u_ref, c_ref, w1_ref, b1_ref, w2t_ref, b2_ref, out_ref, *, bu, ncand, d):
    u = u_ref[...]
    c = c_ref[...]
    w1u = w1_ref[:d, :]
    w1i = w1_ref[d:, :]
    a = jnp.dot(u, w1u, preferred_element_type=jnp.float32) + b1_ref[...]
    t = jnp.dot(c, w1i, preferred_element_type=jnp.float32)
    t = t.reshape(bu, ncand, d)
    h = jnp.tanh(t + a[:, None, :])
    logits = jnp.sum(h * w2t_ref[...][None, :, :], axis=-1) + b2_ref[0, 0]
    out_ref[...] = jax.nn.sigmoid(logits)


def _tc_mlp(u_emb, c_emb, W1, b1, W2, b2, B, NCAND, D, interpret=False):
    BU = 512
    grid = (B // BU,)
    body = functools.partial(_mlp_body, bu=BU, ncand=NCAND, d=D)
    return pl.pallas_call(
        body,
        grid=grid,
        in_specs=[
            pl.BlockSpec((BU, D), lambda i: (i, 0)),
            pl.BlockSpec((BU * NCAND, D), lambda i: (i, 0)),
            pl.BlockSpec((2 * D, D), lambda i: (0, 0)),
            pl.BlockSpec((1, D), lambda i: (0, 0)),
            pl.BlockSpec((1, D), lambda i: (0, 0)),
            pl.BlockSpec((1, 1), lambda i: (0, 0)),
        ],
        out_specs=pl.BlockSpec((BU, NCAND), lambda i: (i, 0)),
        out_shape=jax.ShapeDtypeStruct((B, NCAND), jnp.float32),
        interpret=interpret,
    )(u_emb, c_emb, W1, b1.reshape(1, D), W2.reshape(1, D), b2.reshape(1, 1))


def kernel(user_id, item_id, neg_items, user_table, item_table, W1, b1, W2, b2):
    B = user_id.shape[0]
    NCAND = neg_items.shape[1] + 1
    D = user_table.shape[1]
    uid = user_id.astype(jnp.int32)
    cid = jnp.concatenate(
        [item_id.reshape(-1, 1), neg_items], axis=-1
    ).reshape(-1).astype(jnp.int32)
    NSLICE = 4
    BS = B // NSLICE
    outs = []
    for s in range(NSLICE):
        u_s = lax.dynamic_slice_in_dim(uid, s * BS, BS)
        c_s = lax.dynamic_slice_in_dim(cid, s * BS * NCAND, BS * NCAND)
        u_emb, c_emb = _sc_gather(user_table, item_table, u_s, c_s, BS, NCAND, D)
        outs.append(_tc_mlp(u_emb, c_emb, W1, b1, W2, b2, BS, NCAND, D))
    return jnp.concatenate(outs, axis=0)


# speedup vs baseline: 4.1103x; 1.1334x over previous
import functools

import jax
import jax.numpy as jnp
from jax import lax
from jax.experimental import pallas as pl
from jax.experimental.pallas import tpu as pltpu
from jax.experimental.pallas import tpu_sc as plsc


def _sc_gather(user_table, item_table, uid, cid_even, cid_odd, B, NCAND, D):
    NW = 32
    u_per_w = B // NW
    npairs = (B * NCAND) // 2
    p_per_w = npairs // NW
    CHUNK = 256
    n_chunks = p_per_w // CHUNK

    mesh = plsc.VectorSubcoreMesh(core_axis_name="c", subcore_axis_name="s")

    @functools.partial(
        pl.kernel,
        out_type=(
            jax.ShapeDtypeStruct((B, D), jnp.float32),
            jax.ShapeDtypeStruct((npairs, 2 * D), jnp.float32),
        ),
        mesh=mesh,
        scratch_types=[
            pltpu.VMEM((u_per_w,), jnp.int32),
            pltpu.VMEM((u_per_w, D), jnp.float32),
            pltpu.VMEM((CHUNK,), jnp.int32),
            pltpu.VMEM((CHUNK,), jnp.int32),
            pltpu.VMEM((CHUNK, D), jnp.float32),
            pltpu.VMEM((CHUNK, D), jnp.float32),
            pltpu.VMEM((CHUNK,), jnp.int32),
            pltpu.VMEM((CHUNK,), jnp.int32),
            pltpu.VMEM((CHUNK, D), jnp.float32),
            pltpu.VMEM((CHUNK, D), jnp.float32),
            pltpu.SemaphoreType.DMA,
            pltpu.SemaphoreType.DMA,
            pltpu.SemaphoreType.DMA,
            pltpu.SemaphoreType.DMA,
            pltpu.SemaphoreType.DMA,
        ],
        compiler_params=pltpu.CompilerParams(use_tc_tiling_on_sc=False),
    )
    def k(ut, it, uid_h, ce_h, co_h, uout, cout,
          uidx, ubuf, ie0, io0, bufe0, bufo0, ie1, io1, bufe1, bufo1,
          usem, se0, so0, se1, so1):
        wid = lax.axis_index("s") * 2 + lax.axis_index("c")
        ubase = wid * u_per_w
        pbase = wid * p_per_w
        ies = (ie0, ie1)
        ios = (io0, io1)
        bufes = (bufe0, bufe1)
        bufos = (bufo0, bufo1)
        sems = ((se0, so0), (se1, so1))

        def issue(j, slot):
            off = pbase + j * CHUNK
            pltpu.sync_copy(ce_h.at[pl.ds(off, CHUNK)], ies[slot])
            pltpu.sync_copy(co_h.at[pl.ds(off, CHUNK)], ios[slot])
            ge = pltpu.async_copy(it.at[ies[slot]], bufes[slot], sems[slot][0])
            go = pltpu.async_copy(it.at[ios[slot]], bufos[slot], sems[slot][1])
            return ge, go

        pltpu.sync_copy(uid_h.at[pl.ds(ubase, u_per_w)], uidx)
        ug = pltpu.async_copy(ut.at[uidx], ubuf, usem)
        pending = issue(0, 0)
        ug.wait()
        pltpu.sync_copy(ubuf, uout.at[pl.ds(ubase, u_per_w)])
        for j in range(n_chunks):
            if j + 1 < n_chunks:
                nxt = issue(j + 1, (j + 1) % 2)
            pending[0].wait()
            pending[1].wait()
            off = pbase + j * CHUNK
            pltpu.sync_copy(
                bufes[j % 2], cout.at[pl.ds(off, CHUNK), pl.ds(0, D)]
            )
            pltpu.sync_copy(
                bufos[j % 2], cout.at[pl.ds(off, CHUNK), pl.ds(D, D)]
            )
            if j + 1 < n_chunks:
                pending = nxt

    return k(user_table, item_table, uid, cid_even, cid_odd)


def _mlp_body(u_ref, c2_ref, w1u_ref, w1i2_ref, b1_ref, w2_ref, b2_ref,
              out_ref, *, bu, ncand, d):
    u = u_ref[...]
    c2 = c2_ref[...]
    a = jnp.dot(u, w1u_ref[...], preferred_element_type=jnp.float32) + b1_ref[...]
    ua = jnp.concatenate([a, a], axis=1)
    t = jnp.dot(c2, w1i2_ref[...], preferred_element_type=jnp.float32)
    h = jnp.tanh(t.reshape(bu, ncand // 2, 2 * d) + ua[:, None, :])
    w2 = w2_ref[0][None, None, :]
    le = jnp.sum(h[:, :, :d] * w2, axis=-1)
    lo = jnp.sum(h[:, :, d:] * w2, axis=-1)
    out_ref[...] = jax.nn.sigmoid(
        jnp.concatenate([le, lo], axis=1) + b2_ref[0, 0]
    )


def _tc_mlp(u_emb, c2, W1u, W1i2, b1, w2row, b2, B, NCAND, D, interpret=False):
    BU = 512
    grid = (B // BU,)
    body = functools.partial(_mlp_body, bu=BU, ncand=NCAND, d=D)
    return pl.pallas_call(
        body,
        grid=grid,
        in_specs=[
            pl.BlockSpec((BU, D), lambda i: (i, 0)),
            pl.BlockSpec((BU * NCAND // 2, 2 * D), lambda i: (i, 0)),
            pl.BlockSpec((D, D), lambda i: (0, 0)),
            pl.BlockSpec((2 * D, 2 * D), lambda i: (0, 0)),
            pl.BlockSpec((1, D), lambda i: (0, 0)),
            pl.BlockSpec((1, D), lambda i: (0, 0)),
            pl.BlockSpec((1, 1), lambda i: (0, 0)),
        ],
        out_specs=pl.BlockSpec((BU, NCAND), lambda i: (i, 0)),
        out_shape=jax.ShapeDtypeStruct((B, NCAND), jnp.float32),
        interpret=interpret,
    )(u_emb, c2, W1u, W1i2, b1.reshape(1, D), w2row, b2.reshape(1, 1))


def kernel(user_id, item_id, neg_items, user_table, item_table, W1, b1, W2, b2):
    B = user_id.shape[0]
    NCAND = neg_items.shape[1] + 1
    D = user_table.shape[1]
    uid = user_id.astype(jnp.int32)
    cid = jnp.concatenate(
        [item_id.reshape(-1, 1), neg_items], axis=-1
    ).reshape(-1).astype(jnp.int32)
    cid_even = cid[0::2]
    cid_odd = cid[1::2]
    W1u = W1[:D, :]
    W1i = W1[D:, :]
    zero = jnp.zeros((D, D), jnp.float32)
    W1i2 = jnp.concatenate(
        [jnp.concatenate([W1i, zero], axis=1),
         jnp.concatenate([zero, W1i], axis=1)], axis=0)
    w2row = W2.reshape(1, D)
    NSLICE = 4
    BS = B // NSLICE
    outs = []
    for s in range(NSLICE):
        u_s = lax.dynamic_slice_in_dim(uid, s * BS, BS)
        ce_s = lax.dynamic_slice_in_dim(cid_even, s * BS * NCAND // 2, BS * NCAND // 2)
        co_s = lax.dynamic_slice_in_dim(cid_odd, s * BS * NCAND // 2, BS * NCAND // 2)
        u_emb, c2 = _sc_gather(user_table, item_table, u_s, ce_s, co_s, BS, NCAND, D)
        outs.append(_tc_mlp(u_emb, c2, W1u, W1i2, b1, w2row, b2, BS, NCAND, D))
    cat = jnp.concatenate(outs, axis=0)
    return cat.reshape(B, 2, NCAND // 2).transpose(0, 2, 1).reshape(B, NCAND)


# speedup vs baseline: 4.2045x; 1.0229x over previous
import functools

import jax
import jax.numpy as jnp
from jax import lax
from jax.experimental import pallas as pl
from jax.experimental.pallas import tpu as pltpu
from jax.experimental.pallas import tpu_sc as plsc


def _sc_gather(user_table, item_table, uid_even, uid_odd, cid_even, cid_odd,
               B, NCAND, D):
    NW = 32
    nup = B // 2
    pu_per_w = nup // NW
    npairs = (B * NCAND) // 2
    p_per_w = npairs // NW
    CHUNK = 256
    n_chunks = p_per_w // CHUNK

    mesh = plsc.VectorSubcoreMesh(core_axis_name="c", subcore_axis_name="s")

    @functools.partial(
        pl.kernel,
        out_type=(
            jax.ShapeDtypeStruct((nup, 2 * D), jnp.float32),
            jax.ShapeDtypeStruct((npairs, 2 * D), jnp.float32),
        ),
        mesh=mesh,
        scratch_types=[
            pltpu.VMEM((pu_per_w,), jnp.int32),
            pltpu.VMEM((pu_per_w,), jnp.int32),
            pltpu.VMEM((pu_per_w, D), jnp.float32),
            pltpu.VMEM((pu_per_w, D), jnp.float32),
            pltpu.VMEM((CHUNK,), jnp.int32),
            pltpu.VMEM((CHUNK,), jnp.int32),
            pltpu.VMEM((CHUNK, D), jnp.float32),
            pltpu.VMEM((CHUNK, D), jnp.float32),
            pltpu.VMEM((CHUNK,), jnp.int32),
            pltpu.VMEM((CHUNK,), jnp.int32),
            pltpu.VMEM((CHUNK, D), jnp.float32),
            pltpu.VMEM((CHUNK, D), jnp.float32),
            pltpu.SemaphoreType.DMA,
            pltpu.SemaphoreType.DMA,
            pltpu.SemaphoreType.DMA,
            pltpu.SemaphoreType.DMA,
            pltpu.SemaphoreType.DMA,
            pltpu.SemaphoreType.DMA,
        ],
        compiler_params=pltpu.CompilerParams(use_tc_tiling_on_sc=False),
    )
    def k(ut, it, ue_h, uo_h, ce_h, co_h, uout, cout,
          uide, uido, ubufe, ubufo, ie0, io0, bufe0, bufo0, ie1, io1, bufe1, bufo1,
          useme, usemo, se0, so0, se1, so1):
        wid = lax.axis_index("s") * 2 + lax.axis_index("c")
        pubase = wid * pu_per_w
        pbase = wid * p_per_w
        ies = (ie0, ie1)
        ios = (io0, io1)
        bufes = (bufe0, bufe1)
        bufos = (bufo0, bufo1)
        sems = ((se0, so0), (se1, so1))

        def issue(j, slot):
            off = pbase + j * CHUNK
            pltpu.sync_copy(ce_h.at[pl.ds(off, CHUNK)], ies[slot])
            pltpu.sync_copy(co_h.at[pl.ds(off, CHUNK)], ios[slot])
            ge = pltpu.async_copy(it.at[ies[slot]], bufes[slot], sems[slot][0])
            go = pltpu.async_copy(it.at[ios[slot]], bufos[slot], sems[slot][1])
            return ge, go

        pltpu.sync_copy(ue_h.at[pl.ds(pubase, pu_per_w)], uide)
        pltpu.sync_copy(uo_h.at[pl.ds(pubase, pu_per_w)], uido)
        uge = pltpu.async_copy(ut.at[uide], ubufe, useme)
        ugo = pltpu.async_copy(ut.at[uido], ubufo, usemo)
        pending = issue(0, 0)
        uge.wait()
        ugo.wait()
        pltpu.sync_copy(ubufe, uout.at[pl.ds(pubase, pu_per_w), pl.ds(0, D)])
        pltpu.sync_copy(ubufo, uout.at[pl.ds(pubase, pu_per_w), pl.ds(D, D)])
        for j in range(n_chunks):
            if j + 1 < n_chunks:
                nxt = issue(j + 1, (j + 1) % 2)
            pending[0].wait()
            pending[1].wait()
            off = pbase + j * CHUNK
            pltpu.sync_copy(
                bufes[j % 2], cout.at[pl.ds(off, CHUNK), pl.ds(0, D)]
            )
            pltpu.sync_copy(
                bufos[j % 2], cout.at[pl.ds(off, CHUNK), pl.ds(D, D)]
            )
            if j + 1 < n_chunks:
                pending = nxt

    return k(user_table, item_table, uid_even, uid_odd, cid_even, cid_odd)


def _mlp_body(u2_ref, c2_ref, w1u2_ref, w1i2_ref, b12_ref, w2_ref, b2_ref,
              out_ref, *, bu, ncand, d):
    u2 = u2_ref[...]
    c2 = c2_ref[...]
    a2 = jnp.dot(u2, w1u2_ref[...], preferred_element_type=jnp.float32) + b12_ref[...]
    a = jnp.concatenate([a2[:, :d], a2[:, d:]], axis=0)
    ua = jnp.concatenate([a, a], axis=1)
    t = jnp.dot(c2, w1i2_ref[...], preferred_element_type=jnp.float32)
    h = jnp.tanh(t.reshape(bu, ncand // 2, 2 * d) + ua[:, None, :])
    w2 = w2_ref[0][None, None, :]
    le = jnp.sum(h[:, :, :d] * w2, axis=-1)
    lo = jnp.sum(h[:, :, d:] * w2, axis=-1)
    out_ref[...] = jax.nn.sigmoid(
        jnp.concatenate([le, lo], axis=1) + b2_ref[0, 0]
    )


def _tc_mlp(u2, c2, W1u2, W1i2, b12, w2row, b2, B, NCAND, D, interpret=False):
    BU = 512
    grid = (B // BU,)
    body = functools.partial(_mlp_body, bu=BU, ncand=NCAND, d=D)
    return pl.pallas_call(
        body,
        grid=grid,
        in_specs=[
            pl.BlockSpec((BU // 2, 2 * D), lambda i: (i, 0)),
            pl.BlockSpec((BU * NCAND // 2, 2 * D), lambda i: (i, 0)),
            pl.BlockSpec((2 * D, 2 * D), lambda i: (0, 0)),
            pl.BlockSpec((2 * D, 2 * D), lambda i: (0, 0)),
            pl.BlockSpec((1, 2 * D), lambda i: (0, 0)),
            pl.BlockSpec((1, D), lambda i: (0, 0)),
            pl.BlockSpec((1, 1), lambda i: (0, 0)),
        ],
        out_specs=pl.BlockSpec((BU, NCAND), lambda i: (i, 0)),
        out_shape=jax.ShapeDtypeStruct((B, NCAND), jnp.float32),
        interpret=interpret,
    )(u2, c2, W1u2, W1i2, b12, w2row, b2.reshape(1, 1))


def kernel(user_id, item_id, neg_items, user_table, item_table, W1, b1, W2, b2):
    B = user_id.shape[0]
    NCAND = neg_items.shape[1] + 1
    D = user_table.shape[1]
    uid = user_id.astype(jnp.int32)
    cid = jnp.concatenate(
        [item_id.reshape(-1, 1), neg_items], axis=-1
    ).reshape(-1).astype(jnp.int32)
    cid_even = cid[0::2]
    cid_odd = cid[1::2]
    BU = 512
    W1u = W1[:D, :]
    W1i = W1[D:, :]
    zero = jnp.zeros((D, D), jnp.float32)
    W1u2 = jnp.concatenate(
        [jnp.concatenate([W1u, zero], axis=1),
         jnp.concatenate([zero, W1u], axis=1)], axis=0)
    W1i2 = jnp.concatenate(
        [jnp.concatenate([W1i, zero], axis=1),
         jnp.concatenate([zero, W1i], axis=1)], axis=0)
    b12 = jnp.concatenate([b1, b1]).reshape(1, 2 * D)
    w2row = W2.reshape(1, D)
    NSLICE = 4
    BS = B // NSLICE
    outs = []
    for s in range(NSLICE):
        ur = lax.dynamic_slice_in_dim(uid, s * BS, BS).reshape(BS // BU, 2, BU // 2)
        ue_s = ur[:, 0, :].reshape(-1)
        uo_s = ur[:, 1, :].reshape(-1)
        ce_s = lax.dynamic_slice_in_dim(cid_even, s * BS * NCAND // 2, BS * NCAND // 2)
        co_s = lax.dynamic_slice_in_dim(cid_odd, s * BS * NCAND // 2, BS * NCAND // 2)
        u2, c2 = _sc_gather(user_table, item_table, ue_s, uo_s, ce_s, co_s,
                            BS, NCAND, D)
        outs.append(_tc_mlp(u2, c2, W1u2, W1i2, b12, w2row, b2, BS, NCAND, D))
    cat = jnp.concatenate(outs, axis=0)
    return cat.reshape(B, 2, NCAND // 2).transpose(0, 2, 1).reshape(B, NCAND)


# speedup vs baseline: 5.4886x; 1.3054x over previous
import functools

import jax
import jax.numpy as jnp
from jax import lax
from jax.experimental import pallas as pl
from jax.experimental.pallas import tpu as pltpu
from jax.experimental.pallas import tpu_sc as plsc


def _sc_gather(user_table, item_table, uid, cid, s_off, BS, NCAND, D):
    NW = 32
    nup = BS // 2
    pu_per_w = nup // NW
    npairs = (BS * NCAND) // 2
    p_per_w = npairs // NW
    CHUNK = 256
    n_chunks = p_per_w // CHUNK
    UB = s_off
    CB = s_off * NCAND
    OB = CB + npairs

    mesh = plsc.VectorSubcoreMesh(core_axis_name="c", subcore_axis_name="s")

    @functools.partial(
        pl.kernel,
        out_type=(
            jax.ShapeDtypeStruct((nup, 2 * D), jnp.float32),
            jax.ShapeDtypeStruct((npairs, 2 * D), jnp.float32),
        ),
        mesh=mesh,
        scratch_types=[
            pltpu.VMEM((pu_per_w,), jnp.int32),
            pltpu.VMEM((pu_per_w,), jnp.int32),
            pltpu.VMEM((pu_per_w, D), jnp.float32),
            pltpu.VMEM((pu_per_w, D), jnp.float32),
            pltpu.VMEM((CHUNK,), jnp.int32),
            pltpu.VMEM((CHUNK,), jnp.int32),
            pltpu.VMEM((CHUNK, D), jnp.float32),
            pltpu.VMEM((CHUNK, D), jnp.float32),
            pltpu.VMEM((CHUNK,), jnp.int32),
            pltpu.VMEM((CHUNK,), jnp.int32),
            pltpu.VMEM((CHUNK, D), jnp.float32),
            pltpu.VMEM((CHUNK, D), jnp.float32),
            pltpu.SemaphoreType.DMA,
            pltpu.SemaphoreType.DMA,
            pltpu.SemaphoreType.DMA,
            pltpu.SemaphoreType.DMA,
            pltpu.SemaphoreType.DMA,
            pltpu.SemaphoreType.DMA,
        ],
        compiler_params=pltpu.CompilerParams(use_tc_tiling_on_sc=False),
    )
    def k(ut, it, uid_h, cid_h, uout, cout,
          uide, uido, ubufe, ubufo, ie0, io0, bufe0, bufo0, ie1, io1, bufe1, bufo1,
          useme, usemo, se0, so0, se1, so1):
        wid = lax.axis_index("s") * 2 + lax.axis_index("c")
        pubase = wid * pu_per_w
        pbase = wid * p_per_w
        ies = (ie0, ie1)
        ios = (io0, io1)
        bufes = (bufe0, bufe1)
        bufos = (bufo0, bufo1)
        sems = ((se0, so0), (se1, so1))

        def issue(j, slot):
            off = pbase + j * CHUNK
            pltpu.sync_copy(cid_h.at[pl.ds(CB + off, CHUNK)], ies[slot])
            pltpu.sync_copy(cid_h.at[pl.ds(OB + off, CHUNK)], ios[slot])
            ge = pltpu.async_copy(it.at[ies[slot]], bufes[slot], sems[slot][0])
            go = pltpu.async_copy(it.at[ios[slot]], bufos[slot], sems[slot][1])
            return ge, go

        pltpu.sync_copy(uid_h.at[pl.ds(UB + pubase, pu_per_w)], uide)
        pltpu.sync_copy(uid_h.at[pl.ds(UB + nup + pubase, pu_per_w)], uido)
        uge = pltpu.async_copy(ut.at[uide], ubufe, useme)
        ugo = pltpu.async_copy(ut.at[uido], ubufo, usemo)
        pending = issue(0, 0)
        uge.wait()
        ugo.wait()
        pltpu.sync_copy(ubufe, uout.at[pl.ds(pubase, pu_per_w), pl.ds(0, D)])
        pltpu.sync_copy(ubufo, uout.at[pl.ds(pubase, pu_per_w), pl.ds(D, D)])
        for j in range(n_chunks):
            if j + 1 < n_chunks:
                nxt = issue(j + 1, (j + 1) % 2)
            pending[0].wait()
            pending[1].wait()
            off = pbase + j * CHUNK
            pltpu.sync_copy(
                bufes[j % 2], cout.at[pl.ds(off, CHUNK), pl.ds(0, D)]
            )
            pltpu.sync_copy(
                bufos[j % 2], cout.at[pl.ds(off, CHUNK), pl.ds(D, D)]
            )
            if j + 1 < n_chunks:
                pending = nxt

    return k(user_table, item_table, uid, cid)


def _mlp_body(u2_ref, c2_ref, w1u2_ref, w1i2_ref, b12_ref, w2_ref, b2_ref,
              out_ref, *, bp, ncand, d):
    u2 = u2_ref[...]
    c2 = c2_ref[...]
    a2 = jnp.dot(u2, w1u2_ref[...], preferred_element_type=jnp.float32) + b12_ref[...]
    t = jnp.dot(c2, w1i2_ref[...], preferred_element_type=jnp.float32)
    h = jnp.tanh(t.reshape(bp, ncand, 2 * d) + a2[:, None, :])
    w2 = w2_ref[0][None, None, :]
    le = jnp.sum(h[:, :, :d] * w2, axis=-1)
    lo = jnp.sum(h[:, :, d:] * w2, axis=-1)
    b2v = b2_ref[0, 0]
    out_ref[0] = jax.nn.sigmoid(le + b2v)
    out_ref[1] = jax.nn.sigmoid(lo + b2v)


def _tc_mlp(u2, c2, W1u2, W1i2, b12, w2row, b2, BS, NCAND, D, interpret=False):
    BP = 256
    grid = (BS // 2 // BP,)
    body = functools.partial(_mlp_body, bp=BP, ncand=NCAND, d=D)
    return pl.pallas_call(
        body,
        grid=grid,
        in_specs=[
            pl.BlockSpec((BP, 2 * D), lambda i: (i, 0)),
            pl.BlockSpec((BP * NCAND, 2 * D), lambda i: (i, 0)),
            pl.BlockSpec((2 * D, 2 * D), lambda i: (0, 0)),
            pl.BlockSpec((2 * D, 2 * D), lambda i: (0, 0)),
            pl.BlockSpec((1, 2 * D), lambda i: (0, 0)),
            pl.BlockSpec((1, D), lambda i: (0, 0)),
            pl.BlockSpec((1, 1), lambda i: (0, 0)),
        ],
        out_specs=pl.BlockSpec((2, BP, NCAND), lambda i: (0, i, 0)),
        out_shape=jax.ShapeDtypeStruct((2, BS // 2, NCAND), jnp.float32),
        interpret=interpret,
    )(u2, c2, W1u2, W1i2, b12, w2row, b2.reshape(1, 1))


def kernel(user_id, item_id, neg_items, user_table, item_table, W1, b1, W2, b2):
    B = user_id.shape[0]
    NCAND = neg_items.shape[1] + 1
    D = user_table.shape[1]
    uid = user_id.astype(jnp.int32)
    cid = jnp.concatenate(
        [item_id.reshape(-1, 1), neg_items], axis=-1
    ).reshape(-1).astype(jnp.int32)
    W1u = W1[:D, :]
    W1i = W1[D:, :]
    zero = jnp.zeros((D, D), jnp.float32)
    W1u2 = jnp.concatenate(
        [jnp.concatenate([W1u, zero], axis=1),
         jnp.concatenate([zero, W1u], axis=1)], axis=0)
    W1i2 = jnp.concatenate(
        [jnp.concatenate([W1i, zero], axis=1),
         jnp.concatenate([zero, W1i], axis=1)], axis=0)
    b12 = jnp.concatenate([b1, b1]).reshape(1, 2 * D)
    w2row = W2.reshape(1, D)
    NSLICE = 4
    BS = B // NSLICE
    outs = []
    for s in range(NSLICE):
        u2, c2 = _sc_gather(user_table, item_table, uid, cid, s * BS,
                            BS, NCAND, D)
        o = _tc_mlp(u2, c2, W1u2, W1i2, b12, w2row, b2, BS, NCAND, D)
        outs.append(o.reshape(BS, NCAND))
    return jnp.concatenate(outs, axis=0)
